# baseline (speedup 1.0000x reference)
import numpy as np

import jax
import jax.numpy as jnp
from jax import lax
from jax.experimental import pallas as pl
from jax.experimental.pallas import tpu as pltpu
from jax.experimental.pallas import tpu_sc as plsc

_N = 10000
_E = 320000
_D = 128
_DP = _D // 2

_NTILES = 16
_NW = 2 * _NTILES
_K = 128
_EPT = _E // _NTILES
_SB = 16
_NSB = 10
_NCHUNK = _SB * _NSB
_EPAD = _NCHUNK * _K - _EPT
_WBTILES = 10
_RPT = _N // _WBTILES
_ZROWS = 40
_ZBLKS = _RPT // _ZROWS

_PERM = np.zeros(_D, dtype=np.int32)
for _t in range(4):
    for _i in range(16):
        _PERM[32 * _t + 2 * _i] = 32 * _t + _i
        _PERM[32 * _t + 2 * _i + 1] = 32 * _t + 16 + _i


def _matmul_body(x_ref, w_ref, o_ref):
    pre = jnp.dot(x_ref[...], w_ref[...], preferred_element_type=jnp.float32)
    o_ref[...] = pre.astype(jnp.bfloat16)


def _matmul_packed(x, w_perm):
    blk = 2000
    return pl.pallas_call(
        _matmul_body,
        grid=(_N // blk,),
        in_specs=[
            pl.BlockSpec((blk, _D), lambda i: (i, 0)),
            pl.BlockSpec((_D, _D), lambda i: (0, 0)),
        ],
        out_specs=pl.BlockSpec((blk, _D), lambda i: (i, 0)),
        out_shape=jax.ShapeDtypeStruct((_N, _D), jnp.bfloat16),
    )(x, w_perm)


def _gcn_body(pre_hbm, src_hbm, dst_hbm, w_hbm, out1_hbm, out2_hbm,
              acc, rowsp, rowsf, srcb, dstb, wbuf, lsem, gsem, ssem):
    c = lax.axis_index("c")
    s = lax.axis_index("s")
    cs = c * _NTILES + s

    def _stage(q, qb):
        pltpu.async_copy(src_hbm.at[cs].at[q], srcb.at[qb], lsem.at[qb])
        pltpu.async_copy(dst_hbm.at[cs].at[q], dstb.at[qb], lsem.at[qb])
        pltpu.async_copy(w_hbm.at[cs].at[q], wbuf.at[qb], lsem.at[qb])

    def _stage_wait(q, qb):
        pltpu.make_async_copy(src_hbm.at[cs].at[q], srcb.at[qb],
                              lsem.at[qb]).wait()
        pltpu.make_async_copy(dst_hbm.at[cs].at[q], dstb.at[qb],
                              lsem.at[qb]).wait()
        pltpu.make_async_copy(w_hbm.at[cs].at[q], wbuf.at[qb],
                              lsem.at[qb]).wait()

    _stage(0, 0)

    zeros = jnp.zeros((16,), jnp.float32)

    def _zrow(r, carry):
        for j in range(8):
            rowsf[r, pl.ds(j * 16, 16)] = zeros
        return carry
    lax.fori_loop(0, _ZROWS, _zrow, 0)

    @pl.when(s < _WBTILES)
    def _():
        def _zcp(i, carry):
            pltpu.sync_copy(rowsf.at[pl.ds(0, _ZROWS)],
                            acc.at[pl.ds(s * _RPT + i * _ZROWS, _ZROWS)])
            return carry
        lax.fori_loop(0, _ZBLKS, _zcp, 0)

    _stage_wait(0, 0)
    plsc.subcore_barrier()

    def _issue_gather(qb, m, b):
        pltpu.async_copy(pre_hbm.at[srcb.at[qb].at[m]], rowsp.at[b],
                         gsem.at[b])

    def _wait_gather(b):
        pltpu.make_async_copy(pre_hbm.at[srcb.at[0].at[0]], rowsp.at[b],
                              gsem.at[b]).wait()

    def _scale(qb, m, b):
        def _scale16(k16, carry):
            wv = wbuf[qb, m, pl.ds(k16 * 16, 16)]
            for e in range(16):
                wk = wv[e]
                k = k16 * 16 + e
                for t in range(4):
                    pv = rowsp[b, k, pl.ds(t * 32, 32)]
                    lo, hi = plsc.unpack(
                        pv, format=plsc.PackFormat.INTERLEAVED,
                        preferred_element_type=jnp.float32)
                    rowsf[k, pl.ds(32 * t, 16)] = lo * wk
                    rowsf[k, pl.ds(32 * t + 16, 16)] = hi * wk
            return carry
        lax.fori_loop(0, _K // 16, _scale16, 0)

    _issue_gather(0, 0, 0)

    def _half(j, b, qb, m):
        @pl.when(j < _NCHUNK - 1)
        def _():
            j1 = j + 1
            _issue_gather(lax.rem(lax.div(j1, _SB), 2), lax.rem(j1, _SB),
                          1 - b)

        _wait_gather(b)

        @pl.when(j >= 1)
        def _():
            pltpu.make_async_copy(rowsf, acc.at[dstb.at[0].at[0]],
                                  ssem).wait()

        _scale(qb, m, b)
        pltpu.async_copy(rowsf, acc.at[dstb.at[qb].at[m]], ssem, add=True)

    def _pair(j2, carry):
        j = j2 * 2
        q = lax.div(j, _SB)
        qb = lax.rem(q, 2)
        me = lax.rem(j, _SB)
        _half(j, 0, qb, me)

        @pl.when(jnp.logical_and(me == 0, q < _NSB - 1))
        def _():
            _stage(q + 1, 1 - qb)

        @pl.when(jnp.logical_and(me == _SB - 2, q < _NSB - 1))
        def _():
            _stage_wait(q + 1, 1 - qb)

        _half(j + 1, 1, qb, me + 1)
        return carry
    lax.fori_loop(0, _NCHUNK // 2, _pair, 0)

    pltpu.make_async_copy(rowsf, acc.at[dstb.at[0].at[0]], ssem).wait()
    plsc.subcore_barrier()

    @pl.when(s < _WBTILES)
    def _():
        def _wb(i, carry):
            rb = s * _RPT + i * _ZROWS
            pltpu.sync_copy(acc.at[pl.ds(rb, _ZROWS)],
                            rowsf.at[pl.ds(0, _ZROWS)])

            def _relu_row(r, carry2):
                for j in range(8):
                    sl = (r, pl.ds(j * 16, 16))
                    rowsf[sl] = jnp.maximum(rowsf[sl], 0.0)
                return carry2
            lax.fori_loop(0, _ZROWS, _relu_row, 0)

            @pl.when(c == 0)
            def _():
                pltpu.sync_copy(rowsf.at[pl.ds(0, _ZROWS)],
                                out1_hbm.at[pl.ds(rb, _ZROWS)])

            @pl.when(c == 1)
            def _():
                pltpu.sync_copy(rowsf.at[pl.ds(0, _ZROWS)],
                                out2_hbm.at[pl.ds(rb, _ZROWS)])
            return carry
        lax.fori_loop(0, _ZBLKS, _wb, 0)


def _edge_pass(pre_pk, src, dst, w):
    mesh = plsc.VectorSubcoreMesh(core_axis_name="c", subcore_axis_name="s")
    return pl.kernel(
        _gcn_body,
        out_type=(jax.ShapeDtypeStruct((_N, _D), jnp.float32),
                  jax.ShapeDtypeStruct((_N, _D), jnp.float32)),
        mesh=mesh,
        compiler_params=pltpu.CompilerParams(use_tc_tiling_on_sc=False,
                                             needs_layout_passes=False),
        scratch_types=[
            pltpu.VMEM_SHARED((_N, _D), jnp.float32),
            pltpu.VMEM((2, _K, _D), jnp.bfloat16),
            pltpu.VMEM((_K, _D), jnp.float32),
            pltpu.VMEM((2, _SB, _K), jnp.int32),
            pltpu.VMEM((2, _SB, _K), jnp.int32),
            pltpu.VMEM((2, _SB, _K), jnp.float32),
            pltpu.SemaphoreType.DMA((2,)),
            pltpu.SemaphoreType.DMA((2,)),
            pltpu.SemaphoreType.DMA,
        ],
    )(pre_pk, src, dst, w)


def kernel(x, edge_index, edge_weight, ori_edge_index, ori_edge_weight, W):
    pre_pk = _matmul_packed(x, W[:, _PERM])

    def _prep(a, dtype):
        a = a.astype(dtype).reshape(_NW, _EPT)
        pad = jnp.zeros((_NW, _EPAD), dtype)
        return jnp.concatenate([a, pad], axis=1).reshape(_NW, _NSB, _SB, _K)

    src = _prep(jnp.concatenate([edge_index[0], ori_edge_index[0]]), jnp.int32)
    dst = _prep(jnp.concatenate([edge_index[1], ori_edge_index[1]]), jnp.int32)
    w = _prep(jnp.concatenate([edge_weight, ori_edge_weight]), jnp.float32)
    out1, out2 = _edge_pass(pre_pk, src, dst, w)
    return out1, out2


# speedup vs baseline: 1.3508x; 1.3508x over previous
import numpy as np

import jax
import jax.numpy as jnp
from jax import lax
from jax.experimental import pallas as pl
from jax.experimental.pallas import tpu as pltpu
from jax.experimental.pallas import tpu_sc as plsc

_N = 10000
_E = 320000
_D = 128
_DP = _D // 2

_NTILES = 16
_NW = 2 * _NTILES
_K = 128
_EPT = _E // _NTILES
_SB = 16
_NSB = 10
_NCHUNK = _SB * _NSB
_EPAD = _NCHUNK * _K - _EPT
_WBTILES = 10
_RPT = _N // _WBTILES
_ZROWS = 40
_ZBLKS = _RPT // _ZROWS

_PERM = np.zeros(_D, dtype=np.int32)
for _t in range(4):
    for _i in range(16):
        _PERM[32 * _t + 2 * _i] = 32 * _t + _i
        _PERM[32 * _t + 2 * _i + 1] = 32 * _t + 16 + _i


def _matmul_body(x_ref, w_ref, o_ref):
    pre = jnp.dot(x_ref[...], w_ref[...], preferred_element_type=jnp.float32)
    o_ref[...] = pre.astype(jnp.bfloat16)


def _matmul_packed(x, w_perm):
    blk = 2000
    return pl.pallas_call(
        _matmul_body,
        grid=(_N // blk,),
        in_specs=[
            pl.BlockSpec((blk, _D), lambda i: (i, 0)),
            pl.BlockSpec((_D, _D), lambda i: (0, 0)),
        ],
        out_specs=pl.BlockSpec((blk, _D), lambda i: (i, 0)),
        out_shape=jax.ShapeDtypeStruct((_N, _D), jnp.bfloat16),
    )(x, w_perm)


def _gcn_body(pre_hbm, src_hbm, dst_hbm, w_hbm, out1_hbm, out2_hbm,
              acc, rowsp, rowsf, srcb, dstb, wbuf, lsem, gsem, ssem):
    c = lax.axis_index("c")
    s = lax.axis_index("s")
    cs = c * _NTILES + s

    def _stage(q, qb):
        pltpu.async_copy(src_hbm.at[cs].at[q], srcb.at[qb], lsem.at[qb])
        pltpu.async_copy(dst_hbm.at[cs].at[q], dstb.at[qb], lsem.at[qb])
        pltpu.async_copy(w_hbm.at[cs].at[q], wbuf.at[qb], lsem.at[qb])

    def _stage_wait(q, qb):
        pltpu.make_async_copy(src_hbm.at[cs].at[q], srcb.at[qb],
                              lsem.at[qb]).wait()
        pltpu.make_async_copy(dst_hbm.at[cs].at[q], dstb.at[qb],
                              lsem.at[qb]).wait()
        pltpu.make_async_copy(w_hbm.at[cs].at[q], wbuf.at[qb],
                              lsem.at[qb]).wait()

    _stage(0, 0)

    zeros = jnp.zeros((16,), jnp.float32)

    def _zrow(r, carry):
        for j in range(8):
            rowsf[r, pl.ds(j * 16, 16)] = zeros
        return carry
    lax.fori_loop(0, _ZROWS, _zrow, 0)

    @pl.when(s < _WBTILES)
    def _():
        def _zcp(i, carry):
            pltpu.sync_copy(rowsf.at[pl.ds(0, _ZROWS)],
                            acc.at[pl.ds(s * _RPT + i * _ZROWS, _ZROWS)])
            return carry
        lax.fori_loop(0, _ZBLKS, _zcp, 0)

    _stage_wait(0, 0)
    plsc.subcore_barrier()

    def _issue_gather(qb, m, b):
        pltpu.async_copy(pre_hbm.at[srcb.at[qb].at[m]], rowsp.at[b],
                         gsem.at[b])

    def _wait_gather(b):
        pltpu.make_async_copy(pre_hbm.at[srcb.at[0].at[0]], rowsp.at[b],
                              gsem.at[b]).wait()

    def _scale(qb, m, b):
        @plsc.parallel_loop(0, _K // 16, 1)
        def _scale16(k16):
            wv = wbuf[qb, m, pl.ds(k16 * 16, 16)]
            for e in range(16):
                wk = wv[e]
                k = k16 * 16 + e
                for t in range(4):
                    pv = rowsp[b, k, pl.ds(t * 32, 32)]
                    lo, hi = plsc.unpack(
                        pv, format=plsc.PackFormat.INTERLEAVED,
                        preferred_element_type=jnp.float32)
                    rowsf[k, pl.ds(32 * t, 16)] = lo * wk
                    rowsf[k, pl.ds(32 * t + 16, 16)] = hi * wk

    _issue_gather(0, 0, 0)

    def _half(j, b, qb, m):
        @pl.when(j < _NCHUNK - 1)
        def _():
            j1 = j + 1
            _issue_gather(lax.rem(lax.div(j1, _SB), 2), lax.rem(j1, _SB),
                          1 - b)

        _wait_gather(b)

        @pl.when(j >= 1)
        def _():
            pltpu.make_async_copy(rowsf, acc.at[dstb.at[0].at[0]],
                                  ssem).wait()

        _scale(qb, m, b)
        pltpu.async_copy(rowsf, acc.at[dstb.at[qb].at[m]], ssem, add=True)

    def _pair(j2, carry):
        j = j2 * 2
        q = lax.div(j, _SB)
        qb = lax.rem(q, 2)
        me = lax.rem(j, _SB)
        _half(j, 0, qb, me)

        @pl.when(jnp.logical_and(me == 0, q < _NSB - 1))
        def _():
            _stage(q + 1, 1 - qb)

        @pl.when(jnp.logical_and(me == _SB - 2, q < _NSB - 1))
        def _():
            _stage_wait(q + 1, 1 - qb)

        _half(j + 1, 1, qb, me + 1)
        return carry
    lax.fori_loop(0, _NCHUNK // 2, _pair, 0)

    pltpu.make_async_copy(rowsf, acc.at[dstb.at[0].at[0]], ssem).wait()
    plsc.subcore_barrier()

    @pl.when(s < _WBTILES)
    def _():
        def _wb(i, carry):
            rb = s * _RPT + i * _ZROWS
            pltpu.sync_copy(acc.at[pl.ds(rb, _ZROWS)],
                            rowsf.at[pl.ds(0, _ZROWS)])

            def _relu_row(r, carry2):
                for j in range(8):
                    sl = (r, pl.ds(j * 16, 16))
                    rowsf[sl] = jnp.maximum(rowsf[sl], 0.0)
                return carry2
            lax.fori_loop(0, _ZROWS, _relu_row, 0)

            @pl.when(c == 0)
            def _():
                pltpu.sync_copy(rowsf.at[pl.ds(0, _ZROWS)],
                                out1_hbm.at[pl.ds(rb, _ZROWS)])

            @pl.when(c == 1)
            def _():
                pltpu.sync_copy(rowsf.at[pl.ds(0, _ZROWS)],
                                out2_hbm.at[pl.ds(rb, _ZROWS)])
            return carry
        lax.fori_loop(0, _ZBLKS, _wb, 0)


def _edge_pass(pre_pk, src, dst, w):
    mesh = plsc.VectorSubcoreMesh(core_axis_name="c", subcore_axis_name="s")
    return pl.kernel(
        _gcn_body,
        out_type=(jax.ShapeDtypeStruct((_N, _D), jnp.float32),
                  jax.ShapeDtypeStruct((_N, _D), jnp.float32)),
        mesh=mesh,
        compiler_params=pltpu.CompilerParams(use_tc_tiling_on_sc=False,
                                             needs_layout_passes=False),
        scratch_types=[
            pltpu.VMEM_SHARED((_N, _D), jnp.float32),
            pltpu.VMEM((2, _K, _D), jnp.bfloat16),
            pltpu.VMEM((_K, _D), jnp.float32),
            pltpu.VMEM((2, _SB, _K), jnp.int32),
            pltpu.VMEM((2, _SB, _K), jnp.int32),
            pltpu.VMEM((2, _SB, _K), jnp.float32),
            pltpu.SemaphoreType.DMA((2,)),
            pltpu.SemaphoreType.DMA((2,)),
            pltpu.SemaphoreType.DMA,
        ],
    )(pre_pk, src, dst, w)


def kernel(x, edge_index, edge_weight, ori_edge_index, ori_edge_weight, W):
    pre_pk = _matmul_packed(x, W[:, _PERM])

    def _prep(a, dtype):
        a = a.astype(dtype).reshape(_NW, _EPT)
        pad = jnp.zeros((_NW, _EPAD), dtype)
        return jnp.concatenate([a, pad], axis=1).reshape(_NW, _NSB, _SB, _K)

    src = _prep(jnp.concatenate([edge_index[0], ori_edge_index[0]]), jnp.int32)
    dst = _prep(jnp.concatenate([edge_index[1], ori_edge_index[1]]), jnp.int32)
    w = _prep(jnp.concatenate([edge_weight, ori_edge_weight]), jnp.float32)
    out1, out2 = _edge_pass(pre_pk, src, dst, w)
    return out1, out2


# speedup vs baseline: 1.3538x; 1.0022x over previous
import numpy as np

import jax
import jax.numpy as jnp
from jax import lax
from jax.experimental import pallas as pl
from jax.experimental.pallas import tpu as pltpu
from jax.experimental.pallas import tpu_sc as plsc

_N = 10000
_E = 320000
_D = 128

_NTILES = 16
_NW = 2 * _NTILES
_K = 128
_EPT = _E // _NTILES
_SB = 16
_NSB = 10
_NCHUNK = _SB * _NSB
_EPAD = _NCHUNK * _K - _EPT
_WBTILES = 10
_RPT = _N // _WBTILES
_ZROWS = 40
_ZBLKS = _RPT // _ZROWS

_PERM = np.zeros(_D, dtype=np.int32)
for _t in range(4):
    for _i in range(16):
        _PERM[32 * _t + 2 * _i] = 32 * _t + _i
        _PERM[32 * _t + 2 * _i + 1] = 32 * _t + 16 + _i


def _matmul_body(x_ref, w_ref, o_ref):
    pre = jnp.dot(x_ref[...], w_ref[...], preferred_element_type=jnp.float32)
    o_ref[...] = pre.astype(jnp.bfloat16)


def _matmul_packed(x, w_perm):
    blk = 2000
    return pl.pallas_call(
        _matmul_body,
        grid=(_N // blk,),
        in_specs=[
            pl.BlockSpec((blk, _D), lambda i: (i, 0)),
            pl.BlockSpec((_D, _D), lambda i: (0, 0)),
        ],
        out_specs=pl.BlockSpec((blk, _D), lambda i: (i, 0)),
        out_shape=jax.ShapeDtypeStruct((_N, _D), jnp.bfloat16),
    )(x, w_perm)


def _gcn_body(pre_hbm, src_hbm, dst_hbm, w_hbm, out1_hbm, out2_hbm,
              acc, rowsp, rowsf, srcb, dstb, wbuf, lsem, gsem, ssem):
    c = lax.axis_index("c")
    s = lax.axis_index("s")
    cs = c * _NTILES + s

    def _stage(q, qb):
        pltpu.async_copy(src_hbm.at[cs].at[q], srcb.at[qb], lsem.at[qb])
        pltpu.async_copy(dst_hbm.at[cs].at[q], dstb.at[qb], lsem.at[qb])
        pltpu.async_copy(w_hbm.at[cs].at[q], wbuf.at[qb], lsem.at[qb])

    def _stage_wait(q, qb):
        pltpu.make_async_copy(src_hbm.at[cs].at[q], srcb.at[qb],
                              lsem.at[qb]).wait()
        pltpu.make_async_copy(dst_hbm.at[cs].at[q], dstb.at[qb],
                              lsem.at[qb]).wait()
        pltpu.make_async_copy(w_hbm.at[cs].at[q], wbuf.at[qb],
                              lsem.at[qb]).wait()

    _stage(0, 0)

    zeros = jnp.zeros((16,), jnp.float32)

    def _zrow(r, carry):
        for j in range(8):
            rowsf[r, pl.ds(j * 16, 16)] = zeros
        return carry
    lax.fori_loop(0, _ZROWS, _zrow, 0)

    @pl.when(s < _WBTILES)
    def _():
        def _zcp(i, carry):
            pltpu.sync_copy(rowsf.at[pl.ds(0, _ZROWS)],
                            acc.at[pl.ds(s * _RPT + i * _ZROWS, _ZROWS)])
            return carry
        lax.fori_loop(0, _ZBLKS, _zcp, 0)

    _stage_wait(0, 0)
    plsc.subcore_barrier()

    def _issue_gather(qb, m, b):
        pltpu.async_copy(pre_hbm.at[srcb.at[qb].at[m]], rowsp.at[b],
                         gsem.at[b])

    def _wait_gather(b):
        pltpu.make_async_copy(pre_hbm.at[srcb.at[0].at[0]], rowsp.at[b],
                              gsem.at[b]).wait()

    def _scale(qb, m, b):
        @plsc.parallel_loop(0, _K // 16, 1)
        def _scale16(k16):
            wv = wbuf[qb, m, pl.ds(k16 * 16, 16)]
            for e in range(16):
                wk = wv[e]
                k = k16 * 16 + e
                for t in range(4):
                    pv = rowsp[b, k, pl.ds(t * 32, 32)]
                    lo, hi = plsc.unpack(
                        pv, format=plsc.PackFormat.INTERLEAVED,
                        preferred_element_type=jnp.float32)
                    rowsf[k, pl.ds(32 * t, 16)] = lo * wk
                    rowsf[k, pl.ds(32 * t + 16, 16)] = hi * wk

    _issue_gather(0, 0, 0)

    def _half(j, b, qb, m):
        @pl.when(j < _NCHUNK - 1)
        def _():
            j1 = j + 1
            _issue_gather(lax.rem(lax.div(j1, _SB), 2), lax.rem(j1, _SB),
                          1 - b)

        _wait_gather(b)

        @pl.when(j >= 1)
        def _():
            pltpu.make_async_copy(rowsf, acc.at[dstb.at[0].at[0]],
                                  ssem).wait()

        _scale(qb, m, b)
        pltpu.async_copy(rowsf, acc.at[dstb.at[qb].at[m]], ssem, add=True)

    def _pair(j2, carry):
        j = j2 * 2
        q = lax.div(j, _SB)
        qb = lax.rem(q, 2)
        me = lax.rem(j, _SB)
        _half(j, 0, qb, me)

        @pl.when(jnp.logical_and(me == 0, q < _NSB - 1))
        def _():
            _stage(q + 1, 1 - qb)

        @pl.when(jnp.logical_and(me == _SB - 2, q < _NSB - 1))
        def _():
            _stage_wait(q + 1, 1 - qb)

        _half(j + 1, 1, qb, me + 1)
        return carry
    lax.fori_loop(0, _NCHUNK // 2, _pair, 0)

    pltpu.make_async_copy(rowsf, acc.at[dstb.at[0].at[0]], ssem).wait()
    plsc.subcore_barrier()

    @pl.when(s < _WBTILES)
    def _():
        def _wb(i, carry):
            rb = s * _RPT + i * _ZROWS
            pltpu.sync_copy(acc.at[pl.ds(rb, _ZROWS)],
                            rowsf.at[pl.ds(0, _ZROWS)])

            def _relu_row(r, carry2):
                for j in range(8):
                    sl = (r, pl.ds(j * 16, 16))
                    rowsf[sl] = jnp.maximum(rowsf[sl], 0.0)
                return carry2
            lax.fori_loop(0, _ZROWS, _relu_row, 0)

            @pl.when(c == 0)
            def _():
                pltpu.sync_copy(rowsf.at[pl.ds(0, _ZROWS)],
                                out1_hbm.at[pl.ds(rb, _ZROWS)])

            @pl.when(c == 1)
            def _():
                pltpu.sync_copy(rowsf.at[pl.ds(0, _ZROWS)],
                                out2_hbm.at[pl.ds(rb, _ZROWS)])
            return carry
        lax.fori_loop(0, _ZBLKS, _wb, 0)


def _edge_pass(pre_pk, src, dst, w):
    mesh = plsc.VectorSubcoreMesh(core_axis_name="c", subcore_axis_name="s")
    return pl.kernel(
        _gcn_body,
        out_type=(jax.ShapeDtypeStruct((_N, _D), jnp.float32),
                  jax.ShapeDtypeStruct((_N, _D), jnp.float32)),
        mesh=mesh,
        compiler_params=pltpu.CompilerParams(use_tc_tiling_on_sc=False,
                                             needs_layout_passes=False),
        scratch_types=[
            pltpu.VMEM_SHARED((_N, _D), jnp.float32),
            pltpu.VMEM((2, _K, _D), jnp.bfloat16),
            pltpu.VMEM((_K, _D), jnp.float32),
            pltpu.VMEM((2, _SB, _K), jnp.int32),
            pltpu.VMEM((2, _SB, _K), jnp.int32),
            pltpu.VMEM((2, _SB, _K), jnp.float32),
            pltpu.SemaphoreType.DMA((2,)),
            pltpu.SemaphoreType.DMA((2,)),
            pltpu.SemaphoreType.DMA,
        ],
    )(pre_pk, src, dst, w)


def kernel(x, edge_index, edge_weight, ori_edge_index, ori_edge_weight, W):
    pre_pk = _matmul_packed(x, W[:, _PERM])

    def _prep(a, dtype):
        a = a.astype(dtype).reshape(_NW, _EPT)
        pad = jnp.zeros((_NW, _EPAD), dtype)
        return jnp.concatenate([a, pad], axis=1).reshape(_NW, _NSB, _SB, _K)

    src = _prep(jnp.concatenate([edge_index[0], ori_edge_index[0]]), jnp.int32)
    dst = _prep(jnp.concatenate([edge_index[1], ori_edge_index[1]]), jnp.int32)
    w = _prep(jnp.concatenate([edge_weight, ori_edge_weight]), jnp.float32)
    out1, out2 = _edge_pass(pre_pk, src, dst, w)
    return out1, out2
